# baseline (speedup 1.0000x reference)
import functools

import jax
import jax.numpy as jnp
from jax import lax
from jax.experimental import pallas as pl
from jax.experimental.pallas import tpu as pltpu
from jax.experimental.pallas import tpu_sc as plsc

N_NODES = 10000
N_PAD = 10240
N_EDGES = 320000
E_PAD = 327680
OUT_CH = 47
PAD3 = 64

NC, NS = 2, 16
NW = NC * NS

CHE = 128
ERS = E_PAD // CHE
RPS = ERS // NS
SROWS = N_PAD // NS
ZROWS = 128
NBUF = 5
NBLK = RPS // NBUF

CHD = 80
DRS = N_EDGES // CHD
RPW = DRS // NW

_mesh = plsc.VectorSubcoreMesh(core_axis_name="c", subcore_axis_name="s")
_sc_params = pltpu.CompilerParams(use_tc_tiling_on_sc=False)


def _zero_fill(buf, nrows, width):
    @pl.loop(0, nrows)
    def _(i):
        @pl.loop(0, width // 16)
        def _(j):
            buf[i, pl.ds(j * 16, 16)] = jnp.zeros((16,), jnp.float32)


def _make_edge_pass(width):
    @functools.partial(
        pl.kernel,
        out_type=jax.ShapeDtypeStruct((NC, N_PAD, width), jnp.float32),
        mesh=_mesh,
        scratch_types=[
            pltpu.VMEM_SHARED((N_PAD, width), jnp.float32),
            pltpu.VMEM((RPS, CHE), jnp.int32),
            pltpu.VMEM((RPS, CHE), jnp.int32),
            pltpu.VMEM((NBUF, CHE, width), jnp.float32),
            pltpu.VMEM((ZROWS, width), jnp.float32),
        ]
        + [pltpu.SemaphoreType.DMA] * NBUF,
        compiler_params=_sc_params,
    )
    def edge_pass(src_hbm, dst_hbm, u_hbm, out_hbm, acc, sidx, didx, rows,
                  zbuf, *gsem):
        c = lax.axis_index("c")
        s = lax.axis_index("s")

        _zero_fill(zbuf, ZROWS, width)

        @pl.loop(0, SROWS // ZROWS)
        def _(t):
            pltpu.sync_copy(zbuf, acc.at[pl.ds(s * SROWS + t * ZROWS, ZROWS)])

        plsc.subcore_barrier()

        pltpu.sync_copy(src_hbm.at[pl.ds(s * RPS, RPS)], sidx)
        pltpu.sync_copy(dst_hbm.at[pl.ds(s * RPS, RPS)], didx)

        uc = u_hbm.at[c]
        for b in range(NBUF):
            pltpu.async_copy(uc.at[sidx.at[b]], rows.at[b], gsem[b])

        @pl.loop(0, NBLK)
        def _(t):
            for b in range(NBUF):
                jj = t * NBUF + b
                pltpu.make_async_copy(
                    uc.at[sidx.at[jj]], rows.at[b], gsem[b]
                ).wait()
                pltpu.sync_copy(rows.at[b], acc.at[didx.at[jj]], add=True)

                @pl.when(t < NBLK - 1)
                def _():
                    pltpu.async_copy(
                        uc.at[sidx.at[jj + NBUF]], rows.at[b], gsem[b]
                    )

        plsc.subcore_barrier()

        pltpu.sync_copy(
            acc.at[pl.ds(s * SROWS, SROWS)],
            out_hbm.at[c, pl.ds(s * SROWS, SROWS)],
        )

    return edge_pass


_edge_pass_64 = _make_edge_pass(64)
_edge_pass_32 = _make_edge_pass(PAD3 // 2)


@functools.partial(
    pl.kernel,
    out_type=jax.ShapeDtypeStruct((NC, N_PAD, 16), jnp.float32),
    mesh=_mesh,
    scratch_types=[
        pltpu.VMEM_SHARED((N_PAD, 16), jnp.float32),
        pltpu.VMEM((RPW, CHD), jnp.int32),
        pltpu.VMEM((CHD, 16), jnp.float32),
        pltpu.VMEM((ZROWS, 16), jnp.float32),
        pltpu.SemaphoreType.DMA,
    ],
    compiler_params=_sc_params,
)
def _deg_pass(dst_hbm, out_hbm, acc, didx, ones_v, zbuf, ssem):
    c = lax.axis_index("c")
    s = lax.axis_index("s")
    w = s * NC + c

    _zero_fill(zbuf, ZROWS, 16)

    @pl.loop(0, CHD)
    def _(i):
        ones_v[i, :] = jnp.ones((16,), jnp.float32)

    @pl.loop(0, SROWS // ZROWS)
    def _(t):
        pltpu.sync_copy(zbuf, acc.at[pl.ds(s * SROWS + t * ZROWS, ZROWS)])

    plsc.subcore_barrier()

    pltpu.sync_copy(dst_hbm.at[pl.ds(w * RPW, RPW)], didx)

    @pl.loop(0, RPW // 25)
    def _(t):
        @pl.loop(0, 25)
        def _(j):
            pltpu.async_copy(ones_v, acc.at[didx.at[t * 25 + j]], ssem,
                             add=True)

        @pl.loop(0, 25)
        def _(j):
            pltpu.make_async_copy(ones_v, acc.at[didx.at[0]], ssem).wait()

    plsc.subcore_barrier()

    pltpu.sync_copy(
        acc.at[pl.ds(s * SROWS, SROWS)],
        out_hbm.at[c, pl.ds(s * SROWS, SROWS)],
    )


def _dot(a, b):
    return lax.dot_general(
        a, b, (((1,), (0,)), ((), ())), precision=lax.Precision.HIGHEST
    )


def _split_halves(r, o_ref):
    h = r.shape[1] // 2
    o_ref[0] = r[:, :h]
    o_ref[1] = r[:, h:]


def _tc_matmul(x, W1):
    def body(x_ref, w_ref, o_ref):
        o_ref[...] = _dot(x_ref[...], w_ref[...])

    return pl.pallas_call(
        body,
        out_shape=jax.ShapeDtypeStruct((N_PAD, W1.shape[1]), jnp.float32),
    )(x, W1)


def _tc_first(m1, degp):
    def body(m_ref, dp_ref, ds_ref, u_ref):
        deg = dp_ref[0, :, 0:1] + dp_ref[1, :, 0:1] + 1.0
        ds = lax.rsqrt(deg)
        ds_ref[...] = ds
        _split_halves(m_ref[...] * ds, u_ref)

    return pl.pallas_call(
        body,
        out_shape=(
            jax.ShapeDtypeStruct((N_PAD, 1), jnp.float32),
            jax.ShapeDtypeStruct((NC, N_PAD, m1.shape[1] // 2), jnp.float32),
        ),
    )(m1, degp)


def _tc_mid(p, u, ds, b, W):
    def body(p_ref, u_ref, ds_ref, b_ref, w_ref, o_ref):
        ds_v = ds_ref[...]
        t = jnp.concatenate([p_ref[0] + u_ref[0], p_ref[1] + u_ref[1]], axis=1)
        h = jnp.maximum(ds_v * t + b_ref[...], 0.0)
        _split_halves(_dot(ds_v * h, w_ref[...]), o_ref)

    return pl.pallas_call(
        body,
        out_shape=jax.ShapeDtypeStruct((NC, N_PAD, W.shape[1] // 2), jnp.float32),
    )(p, u, ds, b, W)


def _tc_last(p, u, ds, b):
    def body(p_ref, u_ref, ds_ref, b_ref, o_ref):
        t = jnp.concatenate([p_ref[0] + u_ref[0], p_ref[1] + u_ref[1]], axis=1)
        full = ds_ref[...] * t + b_ref[...]
        o_ref[...] = full[:N_NODES, :OUT_CH]

    return pl.pallas_call(
        body,
        out_shape=jax.ShapeDtypeStruct((N_NODES, OUT_CH), jnp.float32),
    )(p, u, ds, b)


def kernel(x, edge_index, W1, b1, W2, b2, W3, b3):
    ei = edge_index.astype(jnp.int32)
    dummy = jnp.full((E_PAD - N_EDGES,), N_NODES, jnp.int32)
    src = jnp.concatenate([ei[0], dummy]).reshape(ERS, CHE)
    dst = jnp.concatenate([ei[1], dummy]).reshape(ERS, CHE)
    dst_deg = ei[1].reshape(DRS, CHD)
    xp = jnp.pad(x, ((0, N_PAD - N_NODES), (0, 0)))

    degp = _deg_pass(dst_deg)
    m1 = _tc_matmul(xp, W1)
    ds, u1 = _tc_first(m1, degp)

    p1 = _edge_pass_64(src, dst, u1)
    u2 = _tc_mid(p1, u1, ds, b1.reshape(1, -1), W2)

    p2 = _edge_pass_64(src, dst, u2)
    W3p = jnp.pad(W3, ((0, 0), (0, PAD3 - OUT_CH)))
    u3 = _tc_mid(p2, u2, ds, b2.reshape(1, -1), W3p)

    p3 = _edge_pass_32(src, dst, u3)
    out = _tc_last(p3, u3, ds, jnp.pad(b3, (0, PAD3 - OUT_CH)).reshape(1, -1))
    return out


# speedup vs baseline: 2.2518x; 2.2518x over previous
import functools

import jax
import jax.numpy as jnp
from jax import lax
from jax.experimental import pallas as pl
from jax.experimental.pallas import tpu as pltpu
from jax.experimental.pallas import tpu_sc as plsc

N_NODES = 10000
N_EDGES = 320000
OUT_CH = 47
PAD3 = 64

NC, NS = 2, 16
NW = NC * NS
CH = 80
EROWS = N_EDGES // CH
RPW = EROWS // NW
RPS = EROWS // NS
SROWS = N_NODES // NS
ZROWS = 125
NBUF = 5
NBLK = RPS // NBUF
GLEAD = 3

_mesh = plsc.VectorSubcoreMesh(core_axis_name="c", subcore_axis_name="s")
_sc_params = pltpu.CompilerParams(use_tc_tiling_on_sc=False)


def _zero_fill(buf, nrows, width):
    @pl.loop(0, nrows)
    def _(i):
        @pl.loop(0, width // 16)
        def _(j):
            buf[i, pl.ds(j * 16, 16)] = jnp.zeros((16,), jnp.float32)


def _make_edge_pass(width):
    @functools.partial(
        pl.kernel,
        out_type=jax.ShapeDtypeStruct((NC, N_NODES, width), jnp.float32),
        mesh=_mesh,
        scratch_types=[
            pltpu.VMEM_SHARED((N_NODES, width), jnp.float32),
            pltpu.VMEM((RPS, CH), jnp.int32),
            pltpu.VMEM((RPS, CH), jnp.int32),
            pltpu.VMEM((NBUF, CH, width), jnp.float32),
            pltpu.VMEM((ZROWS, width), jnp.float32),
        ]
        + [pltpu.SemaphoreType.DMA] * (2 * NBUF),
        compiler_params=_sc_params,
    )
    def edge_pass(src_hbm, dst_hbm, u_hbm, out_hbm, acc, sidx, didx, rows,
                  zbuf, *sems):
        gsem = sems[:NBUF]
        ssem = sems[NBUF:]
        c = lax.axis_index("c")
        s = lax.axis_index("s")

        _zero_fill(zbuf, ZROWS, width)

        @pl.loop(0, SROWS // ZROWS)
        def _(t):
            pltpu.sync_copy(zbuf, acc.at[pl.ds(s * SROWS + t * ZROWS, ZROWS)])

        plsc.subcore_barrier()

        pltpu.sync_copy(src_hbm.at[pl.ds(s * RPS, RPS)], sidx)
        pltpu.sync_copy(dst_hbm.at[pl.ds(s * RPS, RPS)], didx)

        uc = u_hbm.at[c]
        for b in range(NBUF):
            pltpu.async_copy(uc.at[sidx.at[b]], rows.at[b], gsem[b])

        @pl.loop(0, NBLK)
        def _(t):
            for b in range(NBUF):
                jj = t * NBUF + b
                pltpu.make_async_copy(
                    uc.at[sidx.at[jj]], rows.at[b], gsem[b]
                ).wait()
                pltpu.sync_copy(rows.at[b], acc.at[didx.at[jj]], add=True)

                @pl.when(t < NBLK - 1)
                def _():
                    pltpu.async_copy(
                        uc.at[sidx.at[jj + NBUF]], rows.at[b], gsem[b]
                    )

        plsc.subcore_barrier()

        pltpu.sync_copy(
            acc.at[pl.ds(s * SROWS, SROWS)],
            out_hbm.at[c, pl.ds(s * SROWS, SROWS)],
        )

    return edge_pass


_edge_pass_64 = _make_edge_pass(64)
_edge_pass_32 = _make_edge_pass(PAD3 // 2)


@functools.partial(
    pl.kernel,
    out_type=jax.ShapeDtypeStruct((NC, N_NODES, 16), jnp.float32),
    mesh=_mesh,
    scratch_types=[
        pltpu.VMEM_SHARED((N_NODES, 16), jnp.float32),
        pltpu.VMEM((RPW, CH), jnp.int32),
        pltpu.VMEM((CH, 16), jnp.float32),
        pltpu.VMEM((ZROWS, 16), jnp.float32),
        pltpu.SemaphoreType.DMA,
    ],
    compiler_params=_sc_params,
)
def _deg_pass(dst_hbm, out_hbm, acc, didx, ones_v, zbuf, ssem):
    c = lax.axis_index("c")
    s = lax.axis_index("s")
    w = s * NC + c

    _zero_fill(zbuf, ZROWS, 16)

    @pl.loop(0, CH)
    def _(i):
        ones_v[i, :] = jnp.ones((16,), jnp.float32)

    @pl.loop(0, SROWS // ZROWS)
    def _(t):
        pltpu.sync_copy(zbuf, acc.at[pl.ds(s * SROWS + t * ZROWS, ZROWS)])

    plsc.subcore_barrier()

    pltpu.sync_copy(dst_hbm.at[pl.ds(w * RPW, RPW)], didx)

    @pl.loop(0, RPW // 25)
    def _(t):
        @pl.loop(0, 25)
        def _(j):
            pltpu.async_copy(ones_v, acc.at[didx.at[t * 25 + j]], ssem,
                             add=True)

        @pl.loop(0, 25)
        def _(j):
            pltpu.make_async_copy(ones_v, acc.at[didx.at[0]], ssem).wait()

    plsc.subcore_barrier()

    pltpu.sync_copy(
        acc.at[pl.ds(s * SROWS, SROWS)],
        out_hbm.at[c, pl.ds(s * SROWS, SROWS)],
    )


def _dot(a, b):
    return lax.dot_general(
        a, b, (((1,), (0,)), ((), ())), precision=lax.Precision.HIGHEST
    )


def _split_halves(r, o_ref):
    h = r.shape[1] // 2
    o_ref[0] = r[:, :h]
    o_ref[1] = r[:, h:]


def _tc_matmul(x, W1):
    def body(x_ref, w_ref, o_ref):
        o_ref[...] = _dot(x_ref[...], w_ref[...])

    return pl.pallas_call(
        body,
        out_shape=jax.ShapeDtypeStruct((N_NODES, W1.shape[1]), jnp.float32),
    )(x, W1)


def _tc_first(m1, degp):
    def body(m_ref, dp_ref, ds_ref, u_ref):
        deg = dp_ref[0, :, 0:1] + dp_ref[1, :, 0:1] + 1.0
        ds = lax.rsqrt(deg)
        ds_ref[...] = ds
        _split_halves(m_ref[...] * ds, u_ref)

    return pl.pallas_call(
        body,
        out_shape=(
            jax.ShapeDtypeStruct((N_NODES, 1), jnp.float32),
            jax.ShapeDtypeStruct((NC, N_NODES, m1.shape[1] // 2), jnp.float32),
        ),
    )(m1, degp)


def _tc_mid(p, u, ds, b, W):
    def body(p_ref, u_ref, ds_ref, b_ref, w_ref, o_ref):
        ds_v = ds_ref[...]
        t = jnp.concatenate([p_ref[0] + u_ref[0], p_ref[1] + u_ref[1]], axis=1)
        h = jnp.maximum(ds_v * t + b_ref[...], 0.0)
        _split_halves(_dot(ds_v * h, w_ref[...]), o_ref)

    return pl.pallas_call(
        body,
        out_shape=jax.ShapeDtypeStruct((NC, N_NODES, W.shape[1] // 2), jnp.float32),
    )(p, u, ds, b, W)


def _tc_last(p, u, ds, b):
    def body(p_ref, u_ref, ds_ref, b_ref, o_ref):
        t = jnp.concatenate([p_ref[0] + u_ref[0], p_ref[1] + u_ref[1]], axis=1)
        full = ds_ref[...] * t + b_ref[...]
        o_ref[...] = full[:, :OUT_CH]

    return pl.pallas_call(
        body,
        out_shape=jax.ShapeDtypeStruct((N_NODES, OUT_CH), jnp.float32),
    )(p, u, ds, b)


def kernel(x, edge_index, W1, b1, W2, b2, W3, b3):
    ei = edge_index.astype(jnp.int32)
    src = ei[0].reshape(EROWS, CH)
    dst = ei[1].reshape(EROWS, CH)

    degp = _deg_pass(dst)
    m1 = _tc_matmul(x, W1)
    ds, u1 = _tc_first(m1, degp)

    p1 = _edge_pass_64(src, dst, u1)
    u2 = _tc_mid(p1, u1, ds, b1.reshape(1, -1), W2)

    p2 = _edge_pass_64(src, dst, u2)
    W3p = jnp.pad(W3, ((0, 0), (0, PAD3 - OUT_CH)))
    u3 = _tc_mid(p2, u2, ds, b2.reshape(1, -1), W3p)

    p3 = _edge_pass_32(src, dst, u3)
    out = _tc_last(p3, u3, ds, jnp.pad(b3, (0, PAD3 - OUT_CH)).reshape(1, -1))
    return out
